# baseline (speedup 1.0000x reference)
import functools

import jax
import jax.numpy as jnp
from jax import lax
from jax.experimental import pallas as pl
from jax.experimental.pallas import tpu as pltpu
from jax.experimental.pallas import tpu_sc as plsc

N = 10000
E = 320000
G = 4
D = 128
RB = 1000
NRB = N // RB
EB = 4000
NEB = E // EB
REPEATS = 4


def _enc_body(x_ref, xm_ref, We_ref, be_ref, Ws_ref, Wd_ref,
              h_ref, hs_ref, hd_ref):
    xin = jnp.concatenate([x_ref[...], xm_ref[...]], axis=1)
    h = jnp.maximum(jnp.dot(xin, We_ref[...],
                            preferred_element_type=jnp.float32) + be_ref[...],
                    0.0)
    h_ref[...] = h
    hs_ref[...] = jnp.dot(h, Ws_ref[...], preferred_element_type=jnp.float32)
    hd_ref[...] = jnp.dot(h, Wd_ref[...], preferred_element_type=jnp.float32)


def _encoder(x, xm, W_enc, b_enc, Ws, Wd):
    return pl.pallas_call(
        _enc_body,
        grid=(NRB,),
        in_specs=[
            pl.BlockSpec((RB, 125), lambda i: (i, 0)),
            pl.BlockSpec((RB, 3), lambda i: (i, 0)),
            pl.BlockSpec((128, D), lambda i: (0, 0)),
            pl.BlockSpec((1, D), lambda i: (0, 0)),
            pl.BlockSpec((D, D), lambda i: (0, 0)),
            pl.BlockSpec((D, D), lambda i: (0, 0)),
        ],
        out_specs=[
            pl.BlockSpec((RB, D), lambda i: (i, 0)),
            pl.BlockSpec((RB, D), lambda i: (i, 0)),
            pl.BlockSpec((RB, D), lambda i: (i, 0)),
        ],
        out_shape=[jax.ShapeDtypeStruct((N, D), jnp.float32)] * 3,
    )(x, xm, W_enc, b_enc, Ws, Wd)


def _pool_body(h_ref, batch_ref, bc_ref, xg_ref, xbc_ref, cnt_ref, bcc_ref):
    i = pl.program_id(0)

    @pl.when(i == 0)
    def _():
        xg_ref[...] = jnp.zeros_like(xg_ref)
        xbc_ref[...] = jnp.zeros_like(xbc_ref)
        cnt_ref[...] = jnp.zeros_like(cnt_ref)
        bcc_ref[...] = jnp.zeros_like(bcc_ref)

    onehot = (batch_ref[...] ==
              lax.broadcasted_iota(jnp.int32, (RB, G), 1)).astype(jnp.float32)
    h = h_ref[...]
    bc = bc_ref[...]
    dn = (((0,), (0,)), ((), ()))
    xg_ref[...] += lax.dot_general(onehot, h, dn,
                                   preferred_element_type=jnp.float32)
    xbc_ref[...] += lax.dot_general(onehot * bc, h, dn,
                                    preferred_element_type=jnp.float32)
    ones = jnp.ones((RB, D), jnp.float32)
    cnt_ref[...] += lax.dot_general(onehot, ones, dn,
                                    preferred_element_type=jnp.float32)
    bcc_ref[...] += lax.dot_general(onehot * bc, ones, dn,
                                    preferred_element_type=jnp.float32)


def _pool(h, batch2d, bc2d):
    return pl.pallas_call(
        _pool_body,
        grid=(NRB,),
        in_specs=[
            pl.BlockSpec((RB, D), lambda i: (i, 0)),
            pl.BlockSpec((RB, 1), lambda i: (i, 0)),
            pl.BlockSpec((RB, 1), lambda i: (i, 0)),
        ],
        out_specs=[pl.BlockSpec((G, D), lambda i: (0, 0))] * 4,
        out_shape=[jax.ShapeDtypeStruct((G, D), jnp.float32)] * 4,
    )(h, batch2d, bc2d)


def _eterm_body(ea_ref, We2_ref, bm_ref, out_ref):
    out_ref[...] = jnp.dot(ea_ref[...], We2_ref[...],
                           preferred_element_type=jnp.float32) + bm_ref[...]


def _eterm(edge_attr, We2, b_msg):
    return pl.pallas_call(
        _eterm_body,
        grid=(NEB,),
        in_specs=[
            pl.BlockSpec((EB, 4), lambda i: (i, 0)),
            pl.BlockSpec((4, D), lambda i: (0, 0)),
            pl.BlockSpec((1, D), lambda i: (0, 0)),
        ],
        out_specs=pl.BlockSpec((EB, D), lambda i: (i, 0)),
        out_shape=jax.ShapeDtypeStruct((E, D), jnp.float32),
    )(edge_attr, We2, b_msg)


def _globals_body(xg_ref, cnt_ref, xbc_ref, W3_ref, W4_ref, bu_ref, g_ref):
    x_graph = xg_ref[...] / jnp.maximum(cnt_ref[...], 1e-6)
    g_ref[...] = (jnp.dot(x_graph, W3_ref[...],
                          preferred_element_type=jnp.float32)
                  + jnp.dot(xbc_ref[...], W4_ref[...],
                            preferred_element_type=jnp.float32)
                  + bu_ref[...])


def _globals(xg_sum, cnt, x_BC, W3, W4, b_upd):
    return pl.pallas_call(
        _globals_body,
        out_shape=jax.ShapeDtypeStruct((G, D), jnp.float32),
    )(xg_sum, cnt, x_BC, W3, W4, b_upd)


def _update_body(last, h_ref, a0_ref, a1_ref, batch_ref, g_ref,
                 W1_ref, W2_ref, Ws_ref, Wd_ref, *outs):
    if last:
        hn_ref, = outs
    else:
        hn_ref, hs_ref, hd_ref, xg_ref = outs
    onehot = (batch_ref[...] ==
              lax.broadcasted_iota(jnp.int32, (RB, G), 1)).astype(jnp.float32)
    h = h_ref[...]
    agg = a0_ref[...] + a1_ref[...]
    u = jnp.dot(h, W1_ref[...], preferred_element_type=jnp.float32)
    u += jnp.dot(agg, W2_ref[...], preferred_element_type=jnp.float32)
    u += jnp.dot(onehot, g_ref[...], preferred_element_type=jnp.float32)
    hn = h + jnp.maximum(u, 0.0)
    hn_ref[...] = hn
    if not last:
        hs_ref[...] = jnp.dot(hn, Ws_ref[...],
                              preferred_element_type=jnp.float32)
        hd_ref[...] = jnp.dot(hn, Wd_ref[...],
                              preferred_element_type=jnp.float32)
        i = pl.program_id(0)

        @pl.when(i == 0)
        def _():
            xg_ref[...] = jnp.zeros_like(xg_ref)

        xg_ref[...] += lax.dot_general(onehot, hn, (((0,), (0,)), ((), ())),
                                       preferred_element_type=jnp.float32)


def _update(h, agg0, agg1, batch2d, g, W1, W2, Ws, Wd, last):
    nouts = 1 if last else 4
    out_shape = [jax.ShapeDtypeStruct((N, D), jnp.float32)] * min(nouts, 3)
    out_specs = [pl.BlockSpec((RB, D), lambda i: (i, 0))] * min(nouts, 3)
    if not last:
        out_shape.append(jax.ShapeDtypeStruct((G, D), jnp.float32))
        out_specs.append(pl.BlockSpec((G, D), lambda i: (0, 0)))
    return pl.pallas_call(
        functools.partial(_update_body, last),
        grid=(NRB,),
        in_specs=[
            pl.BlockSpec((RB, D), lambda i: (i, 0)),
            pl.BlockSpec((RB, D), lambda i: (i, 0)),
            pl.BlockSpec((RB, D), lambda i: (i, 0)),
            pl.BlockSpec((RB, 1), lambda i: (i, 0)),
            pl.BlockSpec((G, D), lambda i: (0, 0)),
            pl.BlockSpec((D, D), lambda i: (0, 0)),
            pl.BlockSpec((D, D), lambda i: (0, 0)),
            pl.BlockSpec((D, D), lambda i: (0, 0)),
            pl.BlockSpec((D, D), lambda i: (0, 0)),
        ],
        out_specs=out_specs,
        out_shape=out_shape,
    )(h, agg0, agg1, batch2d, g, W1, W2, Ws, Wd)


def _dec_body(h_ref, Wd_ref, bd_ref, out_ref):
    out_ref[...] = jnp.dot(h_ref[...], Wd_ref[...],
                           preferred_element_type=jnp.float32) + bd_ref[...]


def _decoder(h, W_dec, b_dec):
    return pl.pallas_call(
        _dec_body,
        grid=(NRB,),
        in_specs=[
            pl.BlockSpec((RB, D), lambda i: (i, 0)),
            pl.BlockSpec((D, 4), lambda i: (0, 0)),
            pl.BlockSpec((1, 4), lambda i: (0, 0)),
        ],
        out_specs=pl.BlockSpec((RB, 4), lambda i: (i, 0)),
        out_shape=jax.ShapeDtypeStruct((N, 4), jnp.float32),
    )(h, W_dec, b_dec)


K = 40
EPT = E // 32
NCHUNK = EPT // K
EXP = 624
TAIL = N - 16 * EXP

_sc_mesh = plsc.VectorSubcoreMesh(core_axis_name="c", subcore_axis_name="s")


assert NCHUNK % 4 == 2

@functools.partial(
    pl.kernel,
    out_type=jax.ShapeDtypeStruct((2 * N, D), jnp.float32),
    mesh=_sc_mesh,
    scratch_types=(
        [pltpu.VMEM((K, D), jnp.float32)] * 8
        + [pltpu.VMEM((K,), jnp.int32)] * 8
        + [pltpu.VMEM_SHARED((N, D), jnp.float32)]
        + [pltpu.SemaphoreType.DMA] * 10
    ),
)
def _edge_sc(hs_hbm, hd_hbm, et_hbm, src_hbm, dst_hbm, out_hbm,
             rS0, rD0, rE0, rM0, rS1, rD1, rE1, rM1,
             is0, id0, is1, id1, is2, id2, is3, id3,
             aggsh,
             ga0, gb0, gc0, ga1, gb1, gc1, sd0, sd1, si0, si1):
    c = lax.axis_index("c")
    s = lax.axis_index("s")
    zero16 = jnp.zeros((16,), jnp.float32)

    def zrow(r, carry):
        for j in range(8):
            rS0[r, pl.ds(j * 16, 16)] = zero16
        return carry

    lax.fori_loop(0, K, zrow, 0)
    rowbase = s * EXP
    for i in range(EXP // K):
        pltpu.async_copy(rS0, aggsh.at[pl.ds(rowbase + i * K, K)], ga0)
    pltpu.async_copy(rS0.at[pl.ds(0, EXP - (EXP // K) * K)],
                     aggsh.at[pl.ds(rowbase + (EXP // K) * K,
                                    EXP - (EXP // K) * K)], ga0)

    @pl.when(s == 15)
    def _():
        pltpu.sync_copy(rS0.at[pl.ds(0, TAIL)],
                        aggsh.at[pl.ds(16 * EXP, TAIL)])

    for i in range(EXP // K):
        pltpu.make_async_copy(rS0,
                              aggsh.at[pl.ds(rowbase + i * K, K)], ga0).wait()
    pltpu.make_async_copy(rS0.at[pl.ds(0, EXP - (EXP // K) * K)],
                          aggsh.at[pl.ds(rowbase + (EXP // K) * K,
                                         EXP - (EXP // K) * K)], ga0).wait()

    plsc.subcore_barrier()

    ebase = c * (E // 2) + s * EPT
    dbuf = ((rS0, rD0, rE0, rM0, ga0, gb0, gc0, sd0),
            (rS1, rD1, rE1, rM1, ga1, gb1, gc1, sd1))
    sis = (si0, si1)
    ibuf = ((is0, id0), (is1, id1), (is2, id2), (is3, id3))

    def off_of(cc):
        return pl.multiple_of(ebase + cc * K, 8)

    def idx_load_sync(cc, j):
        pltpu.sync_copy(src_hbm.at[pl.ds(off_of(cc), K)], ibuf[j][0])
        pltpu.sync_copy(dst_hbm.at[pl.ds(off_of(cc), K)], ibuf[j][1])

    def idx_load_async(cc, j, si):
        pltpu.async_copy(src_hbm.at[pl.ds(off_of(cc), K)], ibuf[j][0], si)
        pltpu.async_copy(dst_hbm.at[pl.ds(off_of(cc), K)], ibuf[j][1], si)

    def idx_wait(cc, j, si):
        pltpu.make_async_copy(src_hbm.at[pl.ds(off_of(cc), K)],
                              ibuf[j][0], si).wait()
        pltpu.make_async_copy(dst_hbm.at[pl.ds(off_of(cc), K)],
                              ibuf[j][1], si).wait()

    def gather_issue(cc, j, b):
        rS, rD, rE = dbuf[b][0:3]
        ga, gb, gc = dbuf[b][4:7]
        pltpu.async_copy(hs_hbm.at[ibuf[j][0]], rS, ga)
        pltpu.async_copy(hd_hbm.at[ibuf[j][1]], rD, gb)
        pltpu.async_copy(et_hbm.at[pl.ds(off_of(cc), K)], rE, gc)

    def gather_wait(cc, j, b):
        rS, rD, rE = dbuf[b][0:3]
        ga, gb, gc = dbuf[b][4:7]
        pltpu.make_async_copy(hs_hbm.at[ibuf[j][0]], rS, ga).wait()
        pltpu.make_async_copy(hd_hbm.at[ibuf[j][1]], rD, gb).wait()
        pltpu.make_async_copy(et_hbm.at[pl.ds(off_of(cc), K)], rE, gc).wait()

    def compute(b):
        rS, rD, rE, rM = dbuf[b][0:4]

        @plsc.parallel_loop(0, K, 1, unroll=4)
        def _row(r):
            for j in range(8):
                sl = pl.ds(j * 16, 16)
                rM[r, sl] = jnp.maximum(rS[r, sl] + rD[r, sl] + rE[r, sl],
                                        0.0)

    def scatter_issue(j, b):
        rM, sd = dbuf[b][3], dbuf[b][7]
        pltpu.async_copy(rM, aggsh.at[ibuf[j][1]], sd, add=True)

    def scatter_wait(j, b):
        rM, sd = dbuf[b][3], dbuf[b][7]
        pltpu.make_async_copy(rM, aggsh.at[ibuf[j][1]], sd).wait()

    idx_load_sync(0, 0)
    idx_load_sync(1, 1)
    gather_issue(0, 0, 0)
    gather_issue(1, 1, 1)
    idx_load_async(2, 2, sis[1])

    def quad(i, carry):
        for jj in range(4):
            cc = 4 * i + jj
            b = jj % 2
            gather_wait(cc, jj, b)

            @pl.when(cc >= 1)
            def _():
                scatter_wait((jj - 1) % 4, 1 - b)

            compute(b)
            scatter_issue(jj, b)
            idx_wait(cc + 2, (jj + 2) % 4, sis[1 - b])
            gather_issue(cc + 2, (jj + 2) % 4, b)

            @pl.when(cc + 3 < NCHUNK)
            def _():
                idx_load_async(cc + 3, (jj + 3) % 4, sis[b])
        return carry

    lax.fori_loop(0, (NCHUNK - 2) // 4, quad, 0)
    for jj, cc in ((0, NCHUNK - 2), (1, NCHUNK - 1)):
        gather_wait(cc, jj, jj)
        scatter_wait((jj - 1) % 4, 1 - jj)
        compute(jj)
        scatter_issue(jj, jj)
    scatter_wait(1, 1)
    plsc.subcore_barrier()
    pltpu.sync_copy(aggsh.at[pl.ds(rowbase, EXP)],
                    out_hbm.at[pl.ds(pl.multiple_of(c * N + rowbase, 8), EXP)])

    @pl.when(s == 15)
    def _():
        pltpu.sync_copy(
            aggsh.at[pl.ds(16 * EXP, TAIL)],
            out_hbm.at[pl.ds(pl.multiple_of(c * N + 16 * EXP, 8), TAIL)])


def _edge_pass(hs, hd, eterm, src, dst):
    agg2 = _edge_sc(hs, hd, eterm, src, dst)
    return agg2[:N], agg2[N:]


def kernel(x, x_mask, edge_index, edge_attr, pos, batch,
           W_enc, b_enc, W_msg, b_msg, W_upd, b_upd, W_dec, b_dec):
    del pos
    xm = x_mask.astype(jnp.float32)
    bc2d = xm[:, 2:3]
    batch2d = batch.reshape(N, 1)
    Ws = W_msg[0:D]
    Wd = W_msg[D:2 * D]
    We2 = W_msg[2 * D:]
    W1 = W_upd[0:D]
    W2 = W_upd[D:2 * D]
    W3 = W_upd[2 * D:3 * D]
    W4 = W_upd[3 * D:]
    src = edge_index[0]
    dst = edge_index[1]

    h, hs, hd = _encoder(x, xm, W_enc, b_enc.reshape(1, D), Ws, Wd)
    xg_sum, xbc_sum, cnt, bcc = _pool(h, batch2d, bc2d)
    x_BC = xbc_sum / jnp.maximum(bcc, 1e-6)
    eterm = _eterm(edge_attr, We2, b_msg.reshape(1, D))

    for r in range(REPEATS):
        g = _globals(xg_sum, cnt, x_BC, W3, W4, b_upd.reshape(1, D))
        agg0, agg1 = _edge_pass(hs, hd, eterm, src, dst)
        last = r == REPEATS - 1
        outs = _update(h, agg0, agg1, batch2d, g, W1, W2, Ws, Wd, last)
        if last:
            h, = outs
        else:
            h, hs, hd, xg_sum = outs

    return _decoder(h, W_dec, b_dec.reshape(1, 4))


# speedup vs baseline: 1.0402x; 1.0402x over previous
import functools

import jax
import jax.numpy as jnp
from jax import lax
from jax.experimental import pallas as pl
from jax.experimental.pallas import tpu as pltpu
from jax.experimental.pallas import tpu_sc as plsc

N = 10000
E = 320000
G = 4
D = 128
RB = 1000
NRB = N // RB
EB = 4000
NEB = E // EB
REPEATS = 4


def _enc_body(x_ref, xm_ref, We_ref, be_ref, Ws_ref, Wd_ref,
              h_ref, hs_ref, hd_ref):
    xin = jnp.concatenate([x_ref[...], xm_ref[...]], axis=1)
    h = jnp.maximum(jnp.dot(xin, We_ref[...],
                            preferred_element_type=jnp.float32) + be_ref[...],
                    0.0)
    h_ref[...] = h
    hs_ref[...] = jnp.dot(h, Ws_ref[...], preferred_element_type=jnp.float32)
    hd_ref[...] = jnp.dot(h, Wd_ref[...], preferred_element_type=jnp.float32)


def _encoder(x, xm, W_enc, b_enc, Ws, Wd):
    return pl.pallas_call(
        _enc_body,
        grid=(NRB,),
        in_specs=[
            pl.BlockSpec((RB, 125), lambda i: (i, 0)),
            pl.BlockSpec((RB, 3), lambda i: (i, 0)),
            pl.BlockSpec((128, D), lambda i: (0, 0)),
            pl.BlockSpec((1, D), lambda i: (0, 0)),
            pl.BlockSpec((D, D), lambda i: (0, 0)),
            pl.BlockSpec((D, D), lambda i: (0, 0)),
        ],
        out_specs=[
            pl.BlockSpec((RB, D), lambda i: (i, 0)),
            pl.BlockSpec((RB, D), lambda i: (i, 0)),
            pl.BlockSpec((RB, D), lambda i: (i, 0)),
        ],
        out_shape=[jax.ShapeDtypeStruct((N, D), jnp.float32)] * 3,
    )(x, xm, W_enc, b_enc, Ws, Wd)


def _pool_body(h_ref, batch_ref, bc_ref, xg_ref, xbc_ref, cnt_ref, bcc_ref):
    i = pl.program_id(0)

    @pl.when(i == 0)
    def _():
        xg_ref[...] = jnp.zeros_like(xg_ref)
        xbc_ref[...] = jnp.zeros_like(xbc_ref)
        cnt_ref[...] = jnp.zeros_like(cnt_ref)
        bcc_ref[...] = jnp.zeros_like(bcc_ref)

    onehot = (batch_ref[...] ==
              lax.broadcasted_iota(jnp.int32, (RB, G), 1)).astype(jnp.float32)
    h = h_ref[...]
    bc = bc_ref[...]
    dn = (((0,), (0,)), ((), ()))
    xg_ref[...] += lax.dot_general(onehot, h, dn,
                                   preferred_element_type=jnp.float32)
    xbc_ref[...] += lax.dot_general(onehot * bc, h, dn,
                                    preferred_element_type=jnp.float32)
    ones = jnp.ones((RB, D), jnp.float32)
    cnt_ref[...] += lax.dot_general(onehot, ones, dn,
                                    preferred_element_type=jnp.float32)
    bcc_ref[...] += lax.dot_general(onehot * bc, ones, dn,
                                    preferred_element_type=jnp.float32)


def _pool(h, batch2d, bc2d):
    return pl.pallas_call(
        _pool_body,
        grid=(NRB,),
        in_specs=[
            pl.BlockSpec((RB, D), lambda i: (i, 0)),
            pl.BlockSpec((RB, 1), lambda i: (i, 0)),
            pl.BlockSpec((RB, 1), lambda i: (i, 0)),
        ],
        out_specs=[pl.BlockSpec((G, D), lambda i: (0, 0))] * 4,
        out_shape=[jax.ShapeDtypeStruct((G, D), jnp.float32)] * 4,
    )(h, batch2d, bc2d)


def _eterm_body(ea_ref, We2_ref, bm_ref, out_ref):
    out_ref[...] = jnp.dot(ea_ref[...], We2_ref[...],
                           preferred_element_type=jnp.float32) + bm_ref[...]


def _eterm(edge_attr, We2, b_msg):
    return pl.pallas_call(
        _eterm_body,
        grid=(NEB,),
        in_specs=[
            pl.BlockSpec((EB, 4), lambda i: (i, 0)),
            pl.BlockSpec((4, D), lambda i: (0, 0)),
            pl.BlockSpec((1, D), lambda i: (0, 0)),
        ],
        out_specs=pl.BlockSpec((EB, D), lambda i: (i, 0)),
        out_shape=jax.ShapeDtypeStruct((E, D), jnp.float32),
    )(edge_attr, We2, b_msg)


def _globals_body(xg_ref, cnt_ref, xbc_ref, W3_ref, W4_ref, bu_ref, g_ref):
    x_graph = xg_ref[...] / jnp.maximum(cnt_ref[...], 1e-6)
    g_ref[...] = (jnp.dot(x_graph, W3_ref[...],
                          preferred_element_type=jnp.float32)
                  + jnp.dot(xbc_ref[...], W4_ref[...],
                            preferred_element_type=jnp.float32)
                  + bu_ref[...])


def _globals(xg_sum, cnt, x_BC, W3, W4, b_upd):
    return pl.pallas_call(
        _globals_body,
        out_shape=jax.ShapeDtypeStruct((G, D), jnp.float32),
    )(xg_sum, cnt, x_BC, W3, W4, b_upd)


def _update_body(last, h_ref, a0_ref, a1_ref, batch_ref, g_ref,
                 W1_ref, W2_ref, Ws_ref, Wd_ref, *outs):
    if last:
        hn_ref, = outs
    else:
        hn_ref, hs_ref, hd_ref, xg_ref = outs
    onehot = (batch_ref[...] ==
              lax.broadcasted_iota(jnp.int32, (RB, G), 1)).astype(jnp.float32)
    h = h_ref[...]
    agg = a0_ref[...] + a1_ref[...]
    u = jnp.dot(h, W1_ref[...], preferred_element_type=jnp.float32)
    u += jnp.dot(agg, W2_ref[...], preferred_element_type=jnp.float32)
    u += jnp.dot(onehot, g_ref[...], preferred_element_type=jnp.float32)
    hn = h + jnp.maximum(u, 0.0)
    hn_ref[...] = hn
    if not last:
        hs_ref[...] = jnp.dot(hn, Ws_ref[...],
                              preferred_element_type=jnp.float32)
        hd_ref[...] = jnp.dot(hn, Wd_ref[...],
                              preferred_element_type=jnp.float32)
        i = pl.program_id(0)

        @pl.when(i == 0)
        def _():
            xg_ref[...] = jnp.zeros_like(xg_ref)

        xg_ref[...] += lax.dot_general(onehot, hn, (((0,), (0,)), ((), ())),
                                       preferred_element_type=jnp.float32)


def _update(h, agg0, agg1, batch2d, g, W1, W2, Ws, Wd, last):
    nouts = 1 if last else 4
    out_shape = [jax.ShapeDtypeStruct((N, D), jnp.float32)] * min(nouts, 3)
    out_specs = [pl.BlockSpec((RB, D), lambda i: (i, 0))] * min(nouts, 3)
    if not last:
        out_shape.append(jax.ShapeDtypeStruct((G, D), jnp.float32))
        out_specs.append(pl.BlockSpec((G, D), lambda i: (0, 0)))
    return pl.pallas_call(
        functools.partial(_update_body, last),
        grid=(NRB,),
        in_specs=[
            pl.BlockSpec((RB, D), lambda i: (i, 0)),
            pl.BlockSpec((RB, D), lambda i: (i, 0)),
            pl.BlockSpec((RB, D), lambda i: (i, 0)),
            pl.BlockSpec((RB, 1), lambda i: (i, 0)),
            pl.BlockSpec((G, D), lambda i: (0, 0)),
            pl.BlockSpec((D, D), lambda i: (0, 0)),
            pl.BlockSpec((D, D), lambda i: (0, 0)),
            pl.BlockSpec((D, D), lambda i: (0, 0)),
            pl.BlockSpec((D, D), lambda i: (0, 0)),
        ],
        out_specs=out_specs,
        out_shape=out_shape,
    )(h, agg0, agg1, batch2d, g, W1, W2, Ws, Wd)


def _dec_body(h_ref, Wd_ref, bd_ref, out_ref):
    out_ref[...] = jnp.dot(h_ref[...], Wd_ref[...],
                           preferred_element_type=jnp.float32) + bd_ref[...]


def _decoder(h, W_dec, b_dec):
    return pl.pallas_call(
        _dec_body,
        grid=(NRB,),
        in_specs=[
            pl.BlockSpec((RB, D), lambda i: (i, 0)),
            pl.BlockSpec((D, 4), lambda i: (0, 0)),
            pl.BlockSpec((1, 4), lambda i: (0, 0)),
        ],
        out_specs=pl.BlockSpec((RB, 4), lambda i: (i, 0)),
        out_shape=jax.ShapeDtypeStruct((N, 4), jnp.float32),
    )(h, W_dec, b_dec)


K = 40
EPT = E // 32
NCHUNK = EPT // K
EXP = 624
TAIL = N - 16 * EXP

_sc_mesh = plsc.VectorSubcoreMesh(core_axis_name="c", subcore_axis_name="s")


assert NCHUNK % 4 == 2

@functools.partial(
    pl.kernel,
    out_type=jax.ShapeDtypeStruct((2 * N, D), jnp.float32),
    mesh=_sc_mesh,
    scratch_types=(
        [pltpu.VMEM((K, D), jnp.float32)] * 8
        + [pltpu.VMEM((K,), jnp.int32)] * 8
        + [pltpu.VMEM_SHARED((N, D), jnp.float32)]
        + [pltpu.SemaphoreType.DMA] * 10
    ),
)
def _edge_sc(hs_hbm, hd_hbm, et_hbm, src_hbm, dst_hbm, out_hbm,
             rS0, rD0, rE0, rM0, rS1, rD1, rE1, rM1,
             is0, id0, is1, id1, is2, id2, is3, id3,
             aggsh,
             ga0, gb0, gc0, ga1, gb1, gc1, sd0, sd1, si0, si1):
    c = lax.axis_index("c")
    s = lax.axis_index("s")
    zero16 = jnp.zeros((16,), jnp.float32)

    def zrow(r, carry):
        for j in range(8):
            rS0[r, pl.ds(j * 16, 16)] = zero16
        return carry

    lax.fori_loop(0, K, zrow, 0)
    rowbase = s * EXP
    for i in range(EXP // K):
        pltpu.async_copy(rS0, aggsh.at[pl.ds(rowbase + i * K, K)], ga0)
    pltpu.async_copy(rS0.at[pl.ds(0, EXP - (EXP // K) * K)],
                     aggsh.at[pl.ds(rowbase + (EXP // K) * K,
                                    EXP - (EXP // K) * K)], ga0)

    @pl.when(s == 15)
    def _():
        pltpu.sync_copy(rS0.at[pl.ds(0, TAIL)],
                        aggsh.at[pl.ds(16 * EXP, TAIL)])

    for i in range(EXP // K):
        pltpu.make_async_copy(rS0,
                              aggsh.at[pl.ds(rowbase + i * K, K)], ga0).wait()
    pltpu.make_async_copy(rS0.at[pl.ds(0, EXP - (EXP // K) * K)],
                          aggsh.at[pl.ds(rowbase + (EXP // K) * K,
                                         EXP - (EXP // K) * K)], ga0).wait()

    plsc.subcore_barrier()

    ebase = c * (E // 2) + s * EPT
    dbuf = ((rS0, rD0, rE0, rM0, ga0, gb0, gc0, sd0),
            (rS1, rD1, rE1, rM1, ga1, gb1, gc1, sd1))
    sis = (si0, si1)
    ibuf = ((is0, id0), (is1, id1), (is2, id2), (is3, id3))

    def off_of(cc):
        return pl.multiple_of(ebase + cc * K, 8)

    def idx_load_sync(cc, j):
        pltpu.sync_copy(src_hbm.at[pl.ds(off_of(cc), K)], ibuf[j][0])
        pltpu.sync_copy(dst_hbm.at[pl.ds(off_of(cc), K)], ibuf[j][1])

    def idx_load_async(cc, j, si):
        pltpu.async_copy(src_hbm.at[pl.ds(off_of(cc), K)], ibuf[j][0], si)
        pltpu.async_copy(dst_hbm.at[pl.ds(off_of(cc), K)], ibuf[j][1], si)

    def idx_wait(cc, j, si):
        pltpu.make_async_copy(src_hbm.at[pl.ds(off_of(cc), K)],
                              ibuf[j][0], si).wait()
        pltpu.make_async_copy(dst_hbm.at[pl.ds(off_of(cc), K)],
                              ibuf[j][1], si).wait()

    def gather_issue(cc, j, b):
        rS, rD, rE = dbuf[b][0:3]
        ga, gb, gc = dbuf[b][4:7]
        pltpu.async_copy(hs_hbm.at[ibuf[j][0]], rS, ga)
        pltpu.async_copy(hd_hbm.at[ibuf[j][1]], rD, gb)
        pltpu.async_copy(et_hbm.at[pl.ds(off_of(cc), K)], rE, gc)

    def gather_wait(cc, j, b):
        rS, rD, rE = dbuf[b][0:3]
        ga, gb, gc = dbuf[b][4:7]
        pltpu.make_async_copy(hs_hbm.at[ibuf[j][0]], rS, ga).wait()
        pltpu.make_async_copy(hd_hbm.at[ibuf[j][1]], rD, gb).wait()
        pltpu.make_async_copy(et_hbm.at[pl.ds(off_of(cc), K)], rE, gc).wait()

    def compute(b):
        rS, rD, rE, rM = dbuf[b][0:4]

        def _row(r, carry):
            for j in range(8):
                sl = pl.ds(j * 16, 16)
                rM[r, sl] = jnp.maximum(rS[r, sl] + rD[r, sl] + rE[r, sl],
                                        0.0)
            return carry

        lax.fori_loop(0, K, _row, 0)

    def scatter_issue(j, b):
        rM, sd = dbuf[b][3], dbuf[b][7]
        pltpu.async_copy(rM, aggsh.at[ibuf[j][1]], sd, add=True)

    def scatter_wait(j, b):
        rM, sd = dbuf[b][3], dbuf[b][7]
        pltpu.make_async_copy(rM, aggsh.at[ibuf[j][1]], sd).wait()

    idx_load_sync(0, 0)
    idx_load_sync(1, 1)
    gather_issue(0, 0, 0)
    gather_issue(1, 1, 1)
    idx_load_async(2, 2, sis[1])

    def quad(i, carry):
        for jj in range(4):
            cc = 4 * i + jj
            b = jj % 2
            gather_wait(cc, jj, b)

            @pl.when(cc >= 1)
            def _():
                scatter_wait((jj - 1) % 4, 1 - b)

            compute(b)
            scatter_issue(jj, b)
            idx_wait(cc + 2, (jj + 2) % 4, sis[1 - b])
            gather_issue(cc + 2, (jj + 2) % 4, b)

            @pl.when(cc + 3 < NCHUNK)
            def _():
                idx_load_async(cc + 3, (jj + 3) % 4, sis[b])
        return carry

    lax.fori_loop(0, (NCHUNK - 2) // 4, quad, 0)
    for jj, cc in ((0, NCHUNK - 2), (1, NCHUNK - 1)):
        gather_wait(cc, jj, jj)
        scatter_wait((jj - 1) % 4, 1 - jj)
        compute(jj)
        scatter_issue(jj, jj)
    scatter_wait(1, 1)
    plsc.subcore_barrier()
    pltpu.sync_copy(aggsh.at[pl.ds(rowbase, EXP)],
                    out_hbm.at[pl.ds(pl.multiple_of(c * N + rowbase, 8), EXP)])

    @pl.when(s == 15)
    def _():
        pltpu.sync_copy(
            aggsh.at[pl.ds(16 * EXP, TAIL)],
            out_hbm.at[pl.ds(pl.multiple_of(c * N + 16 * EXP, 8), TAIL)])


def _edge_pass(hs, hd, eterm, src, dst):
    agg2 = _edge_sc(hs, hd, eterm, src, dst)
    return agg2[:N], agg2[N:]


def kernel(x, x_mask, edge_index, edge_attr, pos, batch,
           W_enc, b_enc, W_msg, b_msg, W_upd, b_upd, W_dec, b_dec):
    del pos
    xm = x_mask.astype(jnp.float32)
    bc2d = xm[:, 2:3]
    batch2d = batch.reshape(N, 1)
    Ws = W_msg[0:D]
    Wd = W_msg[D:2 * D]
    We2 = W_msg[2 * D:]
    W1 = W_upd[0:D]
    W2 = W_upd[D:2 * D]
    W3 = W_upd[2 * D:3 * D]
    W4 = W_upd[3 * D:]
    src = edge_index[0]
    dst = edge_index[1]

    h, hs, hd = _encoder(x, xm, W_enc, b_enc.reshape(1, D), Ws, Wd)
    xg_sum, xbc_sum, cnt, bcc = _pool(h, batch2d, bc2d)
    x_BC = xbc_sum / jnp.maximum(bcc, 1e-6)
    eterm = _eterm(edge_attr, We2, b_msg.reshape(1, D))

    for r in range(REPEATS):
        g = _globals(xg_sum, cnt, x_BC, W3, W4, b_upd.reshape(1, D))
        agg0, agg1 = _edge_pass(hs, hd, eterm, src, dst)
        last = r == REPEATS - 1
        outs = _update(h, agg0, agg1, batch2d, g, W1, W2, Ws, Wd, last)
        if last:
            h, = outs
        else:
            h, hs, hd, xg_sum = outs

    return _decoder(h, W_dec, b_dec.reshape(1, 4))
